# baseline (speedup 1.0000x reference)
import functools

import jax
import jax.numpy as jnp
from jax import lax
from jax.experimental import pallas as pl
from jax.experimental.pallas import tpu as pltpu
from jax.experimental.pallas import tpu_sc as plsc

N = 10000
E = 320000
F_IN = 128
HID = 32
C = 40
CP = 48
N_PAD = 10240
NC = 2
NS = 16
NW = NC * NS
EPW = E // NW
CHUNK = 80
NCH = EPW // CHUNK
RPT = N_PAD // NS
LANES = 16
NG = CHUNK // LANES


def _mesh():
  return plsc.VectorSubcoreMesh(
      core_axis_name="c", subcore_axis_name="s",
      num_cores=NC, num_subcores=NS)


def _sc_params():
  return pltpu.CompilerParams(
      use_tc_tiling_on_sc=False, needs_layout_passes=False)


def _splat(buf, e):
  return plsc.load_gather(buf, [jnp.full((LANES,), e, jnp.int32)])


def _sc_deg(dst, ew):
  @functools.partial(
      pl.kernel,
      out_type=jax.ShapeDtypeStruct((NW, N_PAD), jnp.float32),
      mesh=_mesh(),
      compiler_params=_sc_params(),
      scratch_types=[
          pltpu.VMEM((EPW,), jnp.int32),
          pltpu.VMEM((EPW,), jnp.float32),
          pltpu.VMEM((N_PAD,), jnp.float32),
      ],
  )
  def k(dst_hbm, ew_hbm, out_hbm, dst_b, ew_b, acc):
    cid = lax.axis_index("c")
    sid = lax.axis_index("s")
    wid = sid * NC + cid
    zeros = jnp.zeros((LANES,), jnp.float32)

    @pl.loop(0, N_PAD // LANES)
    def _zero(i):
      acc[pl.ds(i * LANES, LANES)] = zeros

    pltpu.sync_copy(dst_hbm.at[pl.ds(wid * EPW, EPW)], dst_b)
    pltpu.sync_copy(ew_hbm.at[pl.ds(wid * EPW, EPW)], ew_b)

    @pl.loop(0, EPW // LANES)
    def _grp(g):
      iv = dst_b[pl.ds(g * LANES, LANES)]
      vv = ew_b[pl.ds(g * LANES, LANES)]
      plsc.addupdate_scatter(acc, [iv], vv)

    pltpu.sync_copy(acc, out_hbm.at[wid])

  return k(dst, ew)


def _tc1(x, W1, deg_part):
  def body(x_ref, w_ref, dp_ref, h_ref, dis_ref):
    h_ref[...] = jnp.dot(x_ref[...], w_ref[...],
                         preferred_element_type=jnp.float32)
    deg = jnp.sum(dp_ref[...], axis=0)
    dis = jnp.where(deg > 0, lax.rsqrt(jnp.where(deg > 0, deg, 1.0)), 0.0)
    dis_ref[...] = dis.reshape(N_PAD // 128, 128)

  return pl.pallas_call(
      body,
      out_shape=[
          jax.ShapeDtypeStruct((N, HID), jnp.float32),
          jax.ShapeDtypeStruct((N_PAD // 128, 128), jnp.float32),
      ],
  )(x, W1, deg_part)


def _sc_mp1(h, src2d, dst2d, ew, dis):
  @functools.partial(
      pl.kernel,
      out_type=[
          jax.ShapeDtypeStruct((NC, N_PAD, HID), jnp.float32),
          jax.ShapeDtypeStruct((E,), jnp.float32),
      ],
      mesh=_mesh(),
      compiler_params=_sc_params(),
      scratch_types=[
          pltpu.VMEM((N_PAD,), jnp.float32),
          pltpu.VMEM((NCH, CHUNK), jnp.int32),
          pltpu.VMEM((NCH, CHUNK), jnp.int32),
          pltpu.VMEM((EPW,), jnp.float32),
          pltpu.VMEM((EPW,), jnp.float32),
          pltpu.VMEM((CHUNK,), jnp.float32),
          pltpu.VMEM((CHUNK, HID), jnp.float32),
          pltpu.VMEM((CHUNK, HID), jnp.float32),
          pltpu.VMEM((CHUNK, HID), jnp.float32),
          pltpu.VMEM_SHARED((N_PAD, HID), jnp.float32),
          pltpu.SemaphoreType.DMA,
          pltpu.SemaphoreType.DMA,
          pltpu.SemaphoreType.DMA,
          pltpu.SemaphoreType.DMA,
          pltpu.SemaphoreType.DMA,
          pltpu.SemaphoreType.DMA,
      ],
  )
  def k(h_hbm, src_hbm, dst_hbm, ew_hbm, dis_hbm, out_hbm, norm_hbm,
        dis_b, src2, dst2, ew_b, norm_b, norm_c, rows0, rows1, rows2, acc,
        g0, g1, g2, s0, s1, s2):
    cid = lax.axis_index("c")
    sid = lax.axis_index("s")
    wid = sid * NC + cid
    pltpu.sync_copy(dis_hbm, dis_b)
    pltpu.sync_copy(src_hbm.at[pl.ds(wid * NCH, NCH)], src2)
    pltpu.sync_copy(dst_hbm.at[pl.ds(wid * NCH, NCH)], dst2)
    pltpu.sync_copy(ew_hbm.at[pl.ds(wid * EPW, EPW)], ew_b)
    zeros = jnp.zeros((LANES,), jnp.float32)

    @pl.loop(0, CHUNK)
    def _zrow(i):
      for j in range(HID // LANES):
        rows0[i, pl.ds(j * LANES, LANES)] = zeros

    @pl.loop(0, RPT // CHUNK)
    def _zacc(j):
      pltpu.sync_copy(rows0, acc.at[pl.ds(sid * RPT + j * CHUNK, CHUNK)])

    plsc.subcore_barrier()
    bufs = (rows0, rows1, rows2)
    gsems = (g0, g1, g2)
    ssems = (s0, s1, s2)

    def gather(ci, b):
      pltpu.async_copy(h_hbm.at[src2.at[ci]], bufs[b], gsems[b])

    def wait_g(ci, b):
      pltpu.make_async_copy(h_hbm.at[src2.at[ci]], bufs[b], gsems[b]).wait()

    def scat(ci, b):
      pltpu.async_copy(bufs[b], acc.at[dst2.at[ci]], ssems[b], add=True)

    def wait_s(b):
      pltpu.make_async_copy(bufs[b], acc.at[dst2.at[0]], ssems[b]).wait()

    def work(ci, b):
      rows = bufs[b]
      ebase = ci * CHUNK
      for g in range(NG):
        sv = src2[ci, pl.ds(g * LANES, LANES)]
        dv = dst2[ci, pl.ds(g * LANES, LANES)]
        wv = ew_b[pl.ds(ebase + g * LANES, LANES)]
        nv = (plsc.load_gather(dis_b, [sv]) * wv *
              plsc.load_gather(dis_b, [dv]))
        norm_c[pl.ds(g * LANES, LANES)] = nv
        norm_b[pl.ds(ebase + g * LANES, LANES)] = nv
        for i in range(LANES):
          e = g * LANES + i
          sp = _splat(norm_c, e)
          for j in range(HID // LANES):
            rows[e, pl.ds(j * LANES, LANES)] = (
                rows[e, pl.ds(j * LANES, LANES)] * sp)

    gather(0, 0)
    gather(1, 1)
    wait_g(0, 0)
    work(0, 0)
    scat(0, 0)
    gather(2, 2)

    @pl.loop(0, (NCH - 2) // 3)
    def _pipe(k3):
      for t in range(3):
        ci = 3 * k3 + 1 + t
        b = (1 + t) % 3
        wait_g(ci, b)
        work(ci, b)
        scat(ci, b)
        wait_s((b + 2) % 3)
        nci = ci + 2

        @pl.when(nci < NCH)
        def _():
          gather(nci, (b + 2) % 3)

    wait_g(NCH - 1, (NCH - 1) % 3)
    work(NCH - 1, (NCH - 1) % 3)
    scat(NCH - 1, (NCH - 1) % 3)
    wait_s((NCH - 2) % 3)
    wait_s((NCH - 1) % 3)

    plsc.subcore_barrier()
    pltpu.sync_copy(norm_b, norm_hbm.at[pl.ds(wid * EPW, EPW)])
    pltpu.sync_copy(acc.at[pl.ds(sid * RPT, RPT)],
                    out_hbm.at[cid, pl.ds(sid * RPT, RPT)])

  return k(h, src2d, dst2d, ew, dis)


def _tc2(part1, b1, W2p):
  def body(p_ref, b_ref, w_ref, o_ref):
    s = p_ref[0] + p_ref[1] + b_ref[...]
    a = jnp.maximum(s, 0.0)
    o_ref[...] = jnp.dot(a, w_ref[...], preferred_element_type=jnp.float32)

  return pl.pallas_call(
      body,
      out_shape=jax.ShapeDtypeStruct((N_PAD, CP), jnp.float32),
  )(part1, b1, W2p)


def _sc_mp2(h2, src2d, dst2d, norm):
  @functools.partial(
      pl.kernel,
      out_type=jax.ShapeDtypeStruct((NC, N_PAD, CP), jnp.float32),
      mesh=_mesh(),
      compiler_params=_sc_params(),
      scratch_types=[
          pltpu.VMEM((NCH, CHUNK), jnp.int32),
          pltpu.VMEM((NCH, CHUNK), jnp.int32),
          pltpu.VMEM((EPW,), jnp.float32),
          pltpu.VMEM((CHUNK, CP), jnp.float32),
          pltpu.VMEM((CHUNK, CP), jnp.float32),
          pltpu.VMEM((CHUNK, CP), jnp.float32),
          pltpu.VMEM_SHARED((N_PAD, CP), jnp.float32),
          pltpu.SemaphoreType.DMA,
          pltpu.SemaphoreType.DMA,
          pltpu.SemaphoreType.DMA,
          pltpu.SemaphoreType.DMA,
          pltpu.SemaphoreType.DMA,
          pltpu.SemaphoreType.DMA,
      ],
  )
  def k(h_hbm, src_hbm, dst_hbm, norm_hbm, out_hbm,
        src2, dst2, norm_b, rows0, rows1, rows2, acc, g0, g1, g2, s0, s1, s2):
    cid = lax.axis_index("c")
    sid = lax.axis_index("s")
    wid = sid * NC + cid
    pltpu.sync_copy(src_hbm.at[pl.ds(wid * NCH, NCH)], src2)
    pltpu.sync_copy(dst_hbm.at[pl.ds(wid * NCH, NCH)], dst2)
    pltpu.sync_copy(norm_hbm.at[pl.ds(wid * EPW, EPW)], norm_b)
    zeros = jnp.zeros((LANES,), jnp.float32)

    @pl.loop(0, CHUNK)
    def _zrow(i):
      for j in range(CP // LANES):
        rows0[i, pl.ds(j * LANES, LANES)] = zeros

    @pl.loop(0, RPT // CHUNK)
    def _zacc(j):
      pltpu.sync_copy(rows0, acc.at[pl.ds(sid * RPT + j * CHUNK, CHUNK)])

    plsc.subcore_barrier()
    bufs = (rows0, rows1, rows2)
    gsems = (g0, g1, g2)
    ssems = (s0, s1, s2)

    def gather(ci, b):
      pltpu.async_copy(h_hbm.at[src2.at[ci]], bufs[b], gsems[b])

    def wait_g(ci, b):
      pltpu.make_async_copy(h_hbm.at[src2.at[ci]], bufs[b], gsems[b]).wait()

    def scat(ci, b):
      pltpu.async_copy(bufs[b], acc.at[dst2.at[ci]], ssems[b], add=True)

    def wait_s(b):
      pltpu.make_async_copy(bufs[b], acc.at[dst2.at[0]], ssems[b]).wait()

    def work(ci, b):
      rows = bufs[b]
      ebase = ci * CHUNK
      for e in range(CHUNK):
        sp = _splat(norm_b, ebase + e)
        for j in range(CP // LANES):
          rows[e, pl.ds(j * LANES, LANES)] = (
              rows[e, pl.ds(j * LANES, LANES)] * sp)

    gather(0, 0)
    gather(1, 1)
    wait_g(0, 0)
    work(0, 0)
    scat(0, 0)
    gather(2, 2)

    @pl.loop(0, (NCH - 2) // 3)
    def _pipe(k3):
      for t in range(3):
        ci = 3 * k3 + 1 + t
        b = (1 + t) % 3
        wait_g(ci, b)
        work(ci, b)
        scat(ci, b)
        wait_s((b + 2) % 3)
        nci = ci + 2

        @pl.when(nci < NCH)
        def _():
          gather(nci, (b + 2) % 3)

    wait_g(NCH - 1, (NCH - 1) % 3)
    work(NCH - 1, (NCH - 1) % 3)
    scat(NCH - 1, (NCH - 1) % 3)
    wait_s((NCH - 2) % 3)
    wait_s((NCH - 1) % 3)

    plsc.subcore_barrier()
    pltpu.sync_copy(acc.at[pl.ds(sid * RPT, RPT)],
                    out_hbm.at[cid, pl.ds(sid * RPT, RPT)])

  return k(h2, src2d, dst2d, norm)


def _tc3(part2, b2p):
  def body(p_ref, b_ref, o_ref):
    s = p_ref[0, :N, :] + p_ref[1, :N, :] + b_ref[...]
    col = lax.broadcasted_iota(jnp.int32, (N, CP), 1)
    m = col < C
    v = jnp.where(m, s, -1e30)
    mx = jnp.max(v, axis=1, keepdims=True)
    ex = jnp.where(m, jnp.exp(v - mx), 0.0)
    lse = jnp.log(jnp.sum(ex, axis=1, keepdims=True))
    o_ref[...] = (v - mx - lse)[:, :C]

  return pl.pallas_call(
      body,
      out_shape=jax.ShapeDtypeStruct((N, C), jnp.float32),
  )(part2, b2p)


def kernel(x, edge_index, edge_attr, W1, b1, W2, b2):
  src = edge_index[0].astype(jnp.int32)
  dst = edge_index[1].astype(jnp.int32)
  src2d = src.reshape(E // CHUNK, CHUNK)
  dst2d = dst.reshape(E // CHUNK, CHUNK)
  ew = edge_attr.astype(jnp.float32)

  deg_part = _sc_deg(dst, ew)
  h, dis2d = _tc1(x, W1, deg_part)
  dis = dis2d.reshape(N_PAD)
  part1, norm = _sc_mp1(h, src2d, dst2d, ew, dis)
  W2p = jnp.pad(W2, ((0, 0), (0, CP - C)))
  h2 = _tc2(part1, b1.reshape(1, HID), W2p)
  part2 = _sc_mp2(h2, src2d, dst2d, norm)
  return _tc3(part2, jnp.pad(b2, (0, CP - C)).reshape(1, CP))


# speedup vs baseline: 1.5188x; 1.5188x over previous
import functools

import jax
import jax.numpy as jnp
from jax import lax
from jax.experimental import pallas as pl
from jax.experimental.pallas import tpu as pltpu
from jax.experimental.pallas import tpu_sc as plsc

N = 10000
E = 320000
F_IN = 128
HID = 32
C = 40
CP = 48
N_PAD = 10240
NC = 2
NS = 16
NW = NC * NS
CHUNK = 80
NCH = 125
EPW = NCH * CHUNK
EP = NW * EPW
PAD_NODE = N_PAD - 1
RPT = N_PAD // NS
LANES = 16
NG = CHUNK // LANES


def _mesh():
  return plsc.VectorSubcoreMesh(
      core_axis_name="c", subcore_axis_name="s",
      num_cores=NC, num_subcores=NS)


def _sc_params():
  return pltpu.CompilerParams(
      use_tc_tiling_on_sc=False, needs_layout_passes=False)


def _vsplat(vec, i):
  idx = jnp.full((LANES,), i, jnp.int32)
  return vec.at[idx].get(mode="promise_in_bounds")


def _tc1(x, W1):
  def body(x_ref, w_ref, h_ref):
    h_ref[...] = jnp.dot(x_ref[...], w_ref[...],
                         preferred_element_type=jnp.float32)

  return pl.pallas_call(
      body,
      out_shape=jax.ShapeDtypeStruct((N, HID), jnp.float32),
  )(x, W1)


def _newton_rsqrt(x):
  xi = plsc.bitcast(x, jnp.int32)
  y = plsc.bitcast(jnp.int32(0x5F3759DF) - (xi >> 1), jnp.float32)
  for _ in range(4):
    y = y * (1.5 - 0.5 * x * y * y)
  return jnp.where(x > 0, y, 0.0)


EPD = EP // NS
HEPD = EPD // 2


def _sc_mp1(h, src2d, dst2d, ew):
  @functools.partial(
      pl.kernel,
      out_type=[
          jax.ShapeDtypeStruct((NC, N_PAD, HID), jnp.float32),
          jax.ShapeDtypeStruct((EP,), jnp.float32),
      ],
      mesh=_mesh(),
      compiler_params=_sc_params(),
      scratch_types=[
          pltpu.VMEM((N_PAD,), jnp.float32),
          pltpu.VMEM((NCH, CHUNK), jnp.int32),
          pltpu.VMEM((NCH, CHUNK), jnp.int32),
          pltpu.VMEM((EPW,), jnp.float32),
          pltpu.VMEM((EPW,), jnp.float32),
          pltpu.VMEM((CHUNK, HID), jnp.float32),
          pltpu.VMEM((CHUNK, HID), jnp.float32),
          pltpu.VMEM((HEPD // CHUNK, CHUNK), jnp.int32),
          pltpu.VMEM((HEPD,), jnp.float32),
          pltpu.VMEM((N_PAD,), jnp.float32),
          pltpu.VMEM((NS, RPT), jnp.float32),
          pltpu.VMEM((RPT,), jnp.float32),
          pltpu.VMEM_SHARED((N_PAD, HID), jnp.float32),
          pltpu.VMEM_SHARED((NS, NS, RPT), jnp.float32),
          pltpu.VMEM_SHARED((N_PAD,), jnp.float32),
          pltpu.SemaphoreType.DMA,
          pltpu.SemaphoreType.DMA,
      ],
  )
  def k(h_hbm, src_hbm, dst_hbm, ew_hbm, out_hbm, norm_hbm,
        dis_b, src2, dst2, ew_b, norm_b, rows0, rows1,
        dstd_b, ewd_b, deg_acc, red_b, dis_sl, acc, stage3, dis_sh,
        sem0, sem1):
    cid = lax.axis_index("c")
    sid = lax.axis_index("s")
    wid = sid * NC + cid
    pltpu.sync_copy(src_hbm.at[pl.ds(wid * NCH, NCH)], src2)
    pltpu.sync_copy(dst_hbm.at[pl.ds(wid * NCH, NCH)], dst2)
    pltpu.sync_copy(ew_hbm.at[pl.ds(wid * EPW, EPW)], ew_b)
    zeros = jnp.zeros((LANES,), jnp.float32)

    @pl.loop(0, CHUNK)
    def _zrow(i):
      for j in range(HID // LANES):
        rows0[i, pl.ds(j * LANES, LANES)] = zeros

    @pl.loop(0, RPT // CHUNK)
    def _zacc(j):
      pltpu.sync_copy(rows0, acc.at[pl.ds(sid * RPT + j * CHUNK, CHUNK)])

    @pl.loop(0, N_PAD // LANES)
    def _zdeg(i):
      deg_acc[pl.ds(i * LANES, LANES)] = zeros

    hrows = HEPD // CHUNK
    for p in range(2):
      pltpu.sync_copy(
          dst_hbm.at[pl.ds(sid * (EPD // CHUNK) + p * hrows, hrows)], dstd_b)
      pltpu.sync_copy(ew_hbm.at[pl.ds(sid * EPD + p * HEPD, HEPD)], ewd_b)

      @pl.loop(0, hrows)
      def _dgrp(r):
        for g in range(NG):
          iv = dstd_b[r, pl.ds(g * LANES, LANES)]
          vv = ewd_b[pl.ds(r * CHUNK + g * LANES, LANES)]
          plsc.addupdate_scatter(deg_acc, [iv], vv)

    @pl.loop(0, NS)
    def _stg(r):
      pltpu.sync_copy(deg_acc.at[pl.ds(r * RPT, RPT)], stage3.at[r, sid])

    plsc.subcore_barrier()
    pltpu.sync_copy(stage3.at[sid], red_b)
    for kk in range(RPT // LANES):
      t = red_b[0, pl.ds(kk * LANES, LANES)]
      for r in range(1, NS):
        t = t + red_b[r, pl.ds(kk * LANES, LANES)]
      dis_sl[pl.ds(kk * LANES, LANES)] = _newton_rsqrt(t)
    pltpu.sync_copy(dis_sl, dis_sh.at[pl.ds(sid * RPT, RPT)])
    plsc.subcore_barrier()
    pltpu.sync_copy(dis_sh, dis_b)

    def gather(ci, rows, sem):
      pltpu.async_copy(h_hbm.at[src2.at[ci]], rows, sem)

    def wait(ci, rows, sem):
      pltpu.make_async_copy(h_hbm.at[src2.at[ci]], rows, sem).wait()

    def work(ci, rows):
      ebase = ci * CHUNK
      for g in range(NG):
        sv = src2[ci, pl.ds(g * LANES, LANES)]
        dv = dst2[ci, pl.ds(g * LANES, LANES)]
        wv = ew_b[pl.ds(ebase + g * LANES, LANES)]
        nv = (plsc.load_gather(dis_b, [sv]) * wv *
              plsc.load_gather(dis_b, [dv]))
        norm_b[pl.ds(ebase + g * LANES, LANES)] = nv
        for i in range(LANES):
          e = g * LANES + i
          sp = _vsplat(nv, i)
          for j in range(HID // LANES):
            rows[e, pl.ds(j * LANES, LANES)] = (
                rows[e, pl.ds(j * LANES, LANES)] * sp)
      pltpu.sync_copy(rows, acc.at[dst2.at[ci]], add=True)

    gather(0, rows0, sem0)

    @pl.loop(0, (NCH - 1) // 2)
    def _pipe(j):
      ci0 = 2 * j
      gather(ci0 + 1, rows1, sem1)
      wait(ci0, rows0, sem0)
      work(ci0, rows0)
      gather(ci0 + 2, rows0, sem0)
      wait(ci0 + 1, rows1, sem1)
      work(ci0 + 1, rows1)

    wait(NCH - 1, rows0, sem0)
    work(NCH - 1, rows0)

    plsc.subcore_barrier()
    pltpu.sync_copy(norm_b, norm_hbm.at[pl.ds(wid * EPW, EPW)])
    pltpu.sync_copy(acc.at[pl.ds(sid * RPT, RPT)],
                    out_hbm.at[cid, pl.ds(sid * RPT, RPT)])

  return k(h, src2d, dst2d, ew)


CPB = 64


def _tc2(part1, b1, W2p):
  def body(p_ref, b_ref, w_ref, o_ref):
    s = p_ref[0] + p_ref[1] + b_ref[...]
    a = jnp.maximum(s, 0.0)
    o_ref[...] = jnp.dot(
        a, w_ref[...], preferred_element_type=jnp.float32
    ).astype(jnp.bfloat16)

  return pl.pallas_call(
      body,
      out_shape=jax.ShapeDtypeStruct((N_PAD, CPB), jnp.bfloat16),
  )(part1, b1, W2p)


def _sc_mp2(h2, src2d, dst2d, norm):
  @functools.partial(
      pl.kernel,
      out_type=jax.ShapeDtypeStruct((NC, N_PAD, CP), jnp.float32),
      mesh=_mesh(),
      compiler_params=_sc_params(),
      scratch_types=[
          pltpu.VMEM((NCH, CHUNK), jnp.int32),
          pltpu.VMEM((NCH, CHUNK), jnp.int32),
          pltpu.VMEM((EPW,), jnp.float32),
          pltpu.VMEM((CHUNK, CPB), jnp.bfloat16),
          pltpu.VMEM((CHUNK, CPB), jnp.bfloat16),
          pltpu.VMEM((CHUNK, CP), jnp.float32),
          pltpu.VMEM_SHARED((N_PAD, CP), jnp.float32),
          pltpu.SemaphoreType.DMA,
          pltpu.SemaphoreType.DMA,
      ],
  )
  def k(h_hbm, src_hbm, dst_hbm, norm_hbm, out_hbm,
        src2, dst2, norm_b, rows0, rows1, rows_f, acc, sem0, sem1):
    cid = lax.axis_index("c")
    sid = lax.axis_index("s")
    wid = sid * NC + cid
    pltpu.sync_copy(src_hbm.at[pl.ds(wid * NCH, NCH)], src2)
    pltpu.sync_copy(dst_hbm.at[pl.ds(wid * NCH, NCH)], dst2)
    pltpu.sync_copy(norm_hbm.at[pl.ds(wid * EPW, EPW)], norm_b)
    zeros = jnp.zeros((LANES,), jnp.float32)

    @pl.loop(0, CHUNK)
    def _zrow(i):
      for j in range(CP // LANES):
        rows_f[i, pl.ds(j * LANES, LANES)] = zeros

    @pl.loop(0, RPT // CHUNK)
    def _zacc(j):
      pltpu.sync_copy(rows_f, acc.at[pl.ds(sid * RPT + j * CHUNK, CHUNK)])

    plsc.subcore_barrier()

    def gather(ci, rows, sem):
      pltpu.async_copy(h_hbm.at[src2.at[ci]], rows, sem)

    def wait(ci, rows, sem):
      pltpu.make_async_copy(h_hbm.at[src2.at[ci]], rows, sem).wait()

    def work(ci, rows):
      ebase = ci * CHUNK
      for g in range(NG):
        nv = norm_b[pl.ds(ebase + g * LANES, LANES)]
        for i in range(LANES):
          e = g * LANES + i
          sp = _vsplat(nv, i)
          v0 = rows[e, pl.ds(0, 2 * LANES)]
          v1 = rows[e, pl.ds(2 * LANES, 2 * LANES)]
          a0, b0 = plsc.unpack(v0, format=plsc.PackFormat.INTERLEAVED)
          a1, _ = plsc.unpack(v1, format=plsc.PackFormat.INTERLEAVED)
          rows_f[e, pl.ds(0, LANES)] = a0 * sp
          rows_f[e, pl.ds(LANES, LANES)] = b0 * sp
          rows_f[e, pl.ds(2 * LANES, LANES)] = a1 * sp
      pltpu.sync_copy(rows_f, acc.at[dst2.at[ci]], add=True)

    gather(0, rows0, sem0)

    @pl.loop(0, (NCH - 1) // 2)
    def _pipe(j):
      ci0 = 2 * j
      gather(ci0 + 1, rows1, sem1)
      wait(ci0, rows0, sem0)
      work(ci0, rows0)
      gather(ci0 + 2, rows0, sem0)
      wait(ci0 + 1, rows1, sem1)
      work(ci0 + 1, rows1)

    wait(NCH - 1, rows0, sem0)
    work(NCH - 1, rows0)

    plsc.subcore_barrier()
    pltpu.sync_copy(acc.at[pl.ds(sid * RPT, RPT)],
                    out_hbm.at[cid, pl.ds(sid * RPT, RPT)])

  return k(h2, src2d, dst2d, norm)


def _tc3(part2, b2p):
  def body(p_ref, b_ref, o_ref):
    s = p_ref[0, :N, :] + p_ref[1, :N, :] + b_ref[...]
    col = lax.broadcasted_iota(jnp.int32, (N, CP), 1)
    m = col < C
    v = jnp.where(m, s, -1e30)
    mx = jnp.max(v, axis=1, keepdims=True)
    ex = jnp.where(m, jnp.exp(v - mx), 0.0)
    lse = jnp.log(jnp.sum(ex, axis=1, keepdims=True))
    o_ref[...] = (v - mx - lse)[:, :C]

  return pl.pallas_call(
      body,
      out_shape=jax.ShapeDtypeStruct((N, C), jnp.float32),
  )(part2, b2p)


def kernel(x, edge_index, edge_attr, W1, b1, W2, b2):
  pad = EP - E
  src = jnp.pad(edge_index[0].astype(jnp.int32), (0, pad))
  dst = jnp.pad(edge_index[1].astype(jnp.int32), (0, pad),
                constant_values=PAD_NODE)
  src2d = src.reshape(EP // CHUNK, CHUNK)
  dst2d = dst.reshape(EP // CHUNK, CHUNK)
  ew = jnp.pad(edge_attr.astype(jnp.float32), (0, pad))

  h = _tc1(x, W1)
  part1, norm = _sc_mp1(h, src2d, dst2d, ew)
  perm = []
  for base in (0, 32):
    for i in range(16):
      perm += [base + i, base + 16 + i]
  W2p = jnp.pad(W2, ((0, 0), (0, CPB - C)))[:, jnp.array(perm)]
  h2 = _tc2(part1, b1.reshape(1, HID), W2p)
  part2 = _sc_mp2(h2, src2d, dst2d, norm)
  return _tc3(part2, jnp.pad(b2, (0, CP - C)).reshape(1, CP))
